# baseline (speedup 1.0000x reference)
import functools

import jax
import jax.numpy as jnp
from jax import lax
from jax.experimental import pallas as pl
from jax.experimental.pallas import tpu as pltpu
from jax.experimental.pallas import tpu_sc as plsc

B = 16384
D = 16
NNEG = 20
NC, NS, L = 2, 16, 16
NW = NC * NS
V = 1000000
BPW = B // NW
GRP = BPW // L

_f32 = jnp.float32
_i32 = jnp.int32


def _sc_body(cidx, xidx, nidxT, cenT, ctxT, pos_hbm, negr_hbm,
             icen, ictx, ineg, cen_pl, ctx_pl, negrow, pos_v, acc,
             sem, sn0, sn1):
    wid = lax.axis_index("s") * NC + lax.axis_index("c")
    base = wid * BPW

    cps = [
        pltpu.async_copy(cidx.at[pl.ds(base, BPW)], icen, sem),
        pltpu.async_copy(xidx.at[pl.ds(base, BPW)], ictx, sem),
    ]
    for j in range(NNEG):
        cps.append(pltpu.async_copy(
            nidxT.at[j, pl.ds(base, BPW)], ineg.at[pl.ds(j * BPW, BPW)], sem))
    for cp in cps:
        cp.wait()

    cps = []
    for d in range(D):
        cps.append(pltpu.async_copy(cenT.at[d].at[icen], cen_pl.at[d], sem))
        cps.append(pltpu.async_copy(ctxT.at[d].at[ictx], ctx_pl.at[d], sem))
    for cp in cps:
        cp.wait()

    def pgroup(g, _):
        o = g * L
        p = cen_pl[0, pl.ds(o, L)] * ctx_pl[0, pl.ds(o, L)]
        for d in range(1, D):
            p = p + cen_pl[d, pl.ds(o, L)] * ctx_pl[d, pl.ds(o, L)]
        pos_v[pl.ds(o, L)] = p
        return _

    lax.fori_loop(0, GRP, pgroup, None)
    pltpu.sync_copy(pos_v, pos_hbm.at[pl.ds(base, BPW)])

    sns = (sn0, sn1)

    def nstream(d, slot):
        return pltpu.make_async_copy(ctxT.at[d].at[ineg], negrow.at[slot],
                                     sns[slot])

    nstream(0, 0).start()
    for d in range(D):
        slot = d % 2
        if d + 1 < D:
            nstream(d + 1, 1 - slot).start()
        nstream(d, slot).wait()
        cvec = [cen_pl[d, pl.ds(g * L, L)] for g in range(GRP)]

        def jbody(j, _, d=d, slot=slot, cvec=cvec):
            jo = j * BPW
            for g in range(GRP):
                nv = negrow[slot, pl.ds(jo + g * L, L)]
                prod = cvec[g] * nv
                if d == 0:
                    acc[j, pl.ds(g * L, L)] = prod
                else:
                    acc[j, pl.ds(g * L, L)] = acc[j, pl.ds(g * L, L)] + prod
            return _

        lax.fori_loop(0, NNEG, jbody, None)

    pltpu.sync_copy(acc, negr_hbm.at[wid])


_sc_main = functools.partial(
    pl.kernel,
    out_type=(
        jax.ShapeDtypeStruct((B,), _f32),
        jax.ShapeDtypeStruct((NW, NNEG, BPW), _f32),
    ),
    mesh=plsc.VectorSubcoreMesh(core_axis_name="c", subcore_axis_name="s"),
    compiler_params=pltpu.CompilerParams(
        needs_layout_passes=False, use_tc_tiling_on_sc=False),
    scratch_types=[
        pltpu.VMEM((BPW,), _i32),
        pltpu.VMEM((BPW,), _i32),
        pltpu.VMEM((NNEG * BPW,), _i32),
        pltpu.VMEM((D, BPW), _f32),
        pltpu.VMEM((D, BPW), _f32),
        pltpu.VMEM((2, NNEG * BPW), _f32),
        pltpu.VMEM((BPW,), _f32),
        pltpu.VMEM((NNEG, BPW), _f32),
        pltpu.SemaphoreType.DMA,
        pltpu.SemaphoreType.DMA,
        pltpu.SemaphoreType.DMA,
    ],
)(_sc_body)


def _tc_body(pos_ref, neg_ref, pos_o, neg_o):
    pos_o[...] = jax.nn.log_sigmoid(pos_ref[...])
    x = neg_ref[...]
    ls = jax.nn.log_sigmoid(-x)
    neg_o[...] = ls.reshape(NW, NNEG, BPW).sum(axis=1)


_tc_epilogue = pl.pallas_call(
    _tc_body,
    out_shape=(
        jax.ShapeDtypeStruct((B // 128, 128), _f32),
        jax.ShapeDtypeStruct((NW, BPW), _f32),
    ),
)


def kernel(center_idx, context_idx, negative_idx, center_table, context_table):
    cidx = center_idx.astype(_i32)
    xidx = context_idx.astype(_i32)
    nidxT = negative_idx.astype(_i32).T
    pos_raw, neg_raw = _sc_main(cidx, xidx, nidxT,
                                center_table.T, context_table.T)
    pos_ls, neg_s = _tc_epilogue(
        pos_raw.reshape(B // 128, 128),
        neg_raw.reshape(NW * NNEG, BPW),
    )
    return pos_ls.reshape(B), neg_s.reshape(B)


# speedup vs baseline: 3.0978x; 3.0978x over previous
import functools

import jax
import jax.numpy as jnp
from jax import lax
from jax.experimental import pallas as pl
from jax.experimental.pallas import tpu as pltpu
from jax.experimental.pallas import tpu_sc as plsc

B = 16384
D = 16
NNEG = 20
NC, NS, L = 2, 16, 16
NW = NC * NS
V = 1000000
V8 = V // 8
BPW = B // NW
E = 16
NCH = BPW // E
GPC = E // L

_f32 = jnp.float32
_i32 = jnp.int32


def _shift3(src, dst, n):
    def body(g, _):
        o = g * L
        dst[pl.ds(o, L)] = lax.shift_right_logical(src[pl.ds(o, L)], 3)
        return _

    lax.fori_loop(0, n // L, body, None)


def _sc_body(cidx, xidx, nidx, ctab, xtab, pos_hbm, negr_hbm,
             idx_c, idx_x, idx_n, idx_c3, idx_x3, idx_n3,
             crows, xrows, nrows, pos_v, neg_v, s0, s1):
    wid = lax.axis_index("s") * NC + lax.axis_index("c")
    base = wid * BPW
    pltpu.sync_copy(cidx.at[pl.ds(base, BPW)], idx_c)
    pltpu.sync_copy(xidx.at[pl.ds(base, BPW)], idx_x)
    pltpu.sync_copy(nidx.at[pl.ds(base * NNEG, BPW * NNEG)], idx_n)
    _shift3(idx_c, idx_c3, BPW)
    _shift3(idx_x, idx_x3, BPW)
    _shift3(idx_n, idx_n3, BPW * NNEG)

    sems = (s0, s1)

    def streams(ch, b):
        sem = sems[b]
        return [
            pltpu.make_async_copy(
                ctab.at[idx_c3.at[pl.ds(ch * E, E)]], crows.at[b], sem),
            pltpu.make_async_copy(
                xtab.at[idx_x3.at[pl.ds(ch * E, E)]], xrows.at[b], sem),
            pltpu.make_async_copy(
                xtab.at[idx_n3.at[pl.ds(ch * E * NNEG, E * NNEG)]],
                nrows.at[b], sem),
        ]

    for b in range(2):
        for cp in streams(b, b):
            cp.start()

    def chunk(it, _):
        for b in range(2):
            ch = it * 2 + b
            for cp in streams(ch, b):
                cp.wait()

            def group(g, _, b=b, ch=ch):
                lg = g * L + lax.iota(_i32, L)
                off = ch * E + g * L
                civ = idx_c[pl.ds(off, L)]
                ccol = lax.shift_left(jnp.bitwise_and(civ, 7), 4)
                xiv = idx_x[pl.ds(off, L)]
                xcol = lax.shift_left(jnp.bitwise_and(xiv, 7), 4)
                cd = [plsc.load_gather(crows.at[b], [lg, ccol + d])
                      for d in range(D)]
                pos = plsc.load_gather(xrows.at[b], [lg, xcol]) * cd[0]
                for d in range(1, D):
                    pos = pos + plsc.load_gather(xrows.at[b], [lg, xcol + d]) * cd[d]
                pos_v[pl.ds(off, L)] = pos
                rowb = lg * NNEG
                for j in range(NNEG):
                    rj = rowb + j
                    njv = plsc.load_gather(idx_n, [ch * E * NNEG + rj])
                    ncol = lax.shift_left(jnp.bitwise_and(njv, 7), 4)
                    acc = plsc.load_gather(nrows.at[b], [rj, ncol]) * cd[0]
                    for d in range(1, D):
                        acc = acc + plsc.load_gather(nrows.at[b], [rj, ncol + d]) * cd[d]
                    neg_v[j, pl.ds(off, L)] = acc
                return _

            lax.fori_loop(0, GPC, group, None)

            nch = it * 2 + b + 2

            @pl.when(nch < NCH)
            def _(nch=nch, b=b):
                for cp in streams(nch, b):
                    cp.start()

        return _

    lax.fori_loop(0, NCH // 2, chunk, None)

    pltpu.sync_copy(pos_v, pos_hbm.at[pl.ds(base, BPW)])
    pltpu.sync_copy(neg_v, negr_hbm.at[wid])


_sc_dots = functools.partial(
    pl.kernel,
    out_type=(
        jax.ShapeDtypeStruct((B,), _f32),
        jax.ShapeDtypeStruct((NW, NNEG, BPW), _f32),
    ),
    mesh=plsc.VectorSubcoreMesh(core_axis_name="c", subcore_axis_name="s"),
    compiler_params=pltpu.CompilerParams(
        needs_layout_passes=False, use_tc_tiling_on_sc=True),
    scratch_types=[
        pltpu.VMEM((BPW,), _i32),
        pltpu.VMEM((BPW,), _i32),
        pltpu.VMEM((BPW * NNEG,), _i32),
        pltpu.VMEM((BPW,), _i32),
        pltpu.VMEM((BPW,), _i32),
        pltpu.VMEM((BPW * NNEG,), _i32),
        pltpu.VMEM((2, E, 128), _f32),
        pltpu.VMEM((2, E, 128), _f32),
        pltpu.VMEM((2, E * NNEG, 128), _f32),
        pltpu.VMEM((BPW,), _f32),
        pltpu.VMEM((NNEG, BPW), _f32),
        pltpu.SemaphoreType.DMA,
        pltpu.SemaphoreType.DMA,
    ],
)(_sc_body)


def _tc_body(pos_ref, neg_ref, pos_o, neg_o):
    pos_o[...] = jax.nn.log_sigmoid(pos_ref[...])
    x = neg_ref[...]
    ls = jax.nn.log_sigmoid(-x)
    neg_o[...] = ls.reshape(NW, NNEG, BPW).sum(axis=1)


_tc_epilogue = pl.pallas_call(
    _tc_body,
    out_shape=(
        jax.ShapeDtypeStruct((B // 128, 128), _f32),
        jax.ShapeDtypeStruct((NW, BPW), _f32),
    ),
)


def kernel(center_idx, context_idx, negative_idx, center_table, context_table):
    cidx = center_idx.astype(_i32)
    xidx = context_idx.astype(_i32)
    nidx = negative_idx.astype(_i32).reshape(B * NNEG)
    pos_raw, neg_raw = _sc_dots(cidx, xidx, nidx,
                                center_table.reshape(V8, 8 * D),
                                context_table.reshape(V8, 8 * D))
    pos_ls, neg_s = _tc_epilogue(
        pos_raw.reshape(B // 128, 128),
        neg_raw.reshape(NW * NNEG, BPW),
    )
    return pos_ls.reshape(B), neg_s.reshape(B)
